# baseline (device time: 17411 ns/iter reference)
import jax
import jax.numpy as jnp
from jax import lax
from jax.experimental import pallas as pl
from jax.experimental.pallas import tpu as pltpu

N_DEV = 4

A1, A2, A3, B1, B2, B3 = range(6)


def kernel(x, w_mat):
    m, k_per = x.shape
    _, n = w_mat.shape
    m_out = m // N_DEV
    half = n // 2

    col_a = pl.ds(0, half)
    col_b = pl.ds(half, half)

    def body(x_ref, w_ref, out_ref, partial_ref, w_bf,
             send_ref, recv_ref, out_vmem, out_copy_sems,
             send_sems, recv_sems):
        my = lax.axis_index("i")
        left = lax.rem(my + (N_DEV - 1), N_DEV)
        right = lax.rem(my + 1, N_DEV)

        barrier_sem = pltpu.get_barrier_semaphore()
        for nbr in (left, right):
            pl.semaphore_signal(
                barrier_sem, inc=1,
                device_id=(nbr,), device_id_type=pl.DeviceIdType.MESH,
            )

        w_bf[...] = w_ref[...].astype(jnp.bfloat16)

        def rows(c):
            return pl.ds(lax.rem(my + c, N_DEV) * m_out, m_out)

        def gemm_chunk(c):
            partial_ref[rows(c), :] = jnp.dot(
                x_ref[rows(c), :].astype(jnp.bfloat16), w_bf[...],
                preferred_element_type=jnp.float32,
            )

        msgs = {}

        def send(i, tile, tgt):
            send_ref[i, :, :] = tile.astype(jnp.bfloat16)
            msgs[i] = pltpu.make_async_remote_copy(
                src_ref=send_ref.at[i],
                dst_ref=recv_ref.at[i],
                send_sem=send_sems.at[i],
                recv_sem=recv_sems.at[i],
                device_id=(tgt,),
                device_id_type=pl.DeviceIdType.MESH,
            )
            msgs[i].start()

        gemm_chunk(2)
        pl.semaphore_wait(barrier_sem, 2)
        send(A2, partial_ref[rows(2), col_a], left)
        send(B2, partial_ref[rows(2), col_b], right)

        gemm_chunk(1)
        send(A1, partial_ref[rows(1), col_a], right)
        gemm_chunk(3)
        send(B1, partial_ref[rows(3), col_b], left)
        gemm_chunk(0)

        msgs[A2].wait_recv()
        send(A3, recv_ref[A2].astype(jnp.float32)
             + partial_ref[rows(3), col_a], left)

        msgs[B2].wait_recv()
        send(B3, recv_ref[B2].astype(jnp.float32)
             + partial_ref[rows(1), col_b], right)

        msgs[A1].wait_recv()
        msgs[A3].wait_recv()
        y = (
            partial_ref[rows(0), col_a]
            + recv_ref[A1].astype(jnp.float32)
            + recv_ref[A3].astype(jnp.float32)
        )
        out_vmem[:, col_a] = y * jax.nn.sigmoid(y)
        cp_a = pltpu.make_async_copy(
            out_vmem.at[:, col_a], out_ref.at[:, col_a],
            out_copy_sems.at[0],
        )
        cp_a.start()

        msgs[B1].wait_recv()
        msgs[B3].wait_recv()
        y = (
            partial_ref[rows(0), col_b]
            + recv_ref[B1].astype(jnp.float32)
            + recv_ref[B3].astype(jnp.float32)
        )
        out_vmem[:, col_b] = y * jax.nn.sigmoid(y)
        cp_b = pltpu.make_async_copy(
            out_vmem.at[:, col_b], out_ref.at[:, col_b],
            out_copy_sems.at[1],
        )
        cp_b.start()

        cp_a.wait()
        cp_b.wait()
        for r in msgs.values():
            r.wait_send()

    return pl.pallas_call(
        body,
        out_shape=jax.ShapeDtypeStruct((m_out, n), jnp.float32),
        in_specs=[
            pl.BlockSpec(memory_space=pltpu.VMEM),
            pl.BlockSpec(memory_space=pltpu.VMEM),
        ],
        out_specs=pl.BlockSpec(memory_space=pltpu.MemorySpace.HBM),
        scratch_shapes=[
            pltpu.VMEM((m, n), jnp.float32),
            pltpu.VMEM((k_per, n), jnp.bfloat16),
            pltpu.VMEM((6, m_out, half), jnp.bfloat16),
            pltpu.VMEM((6, m_out, half), jnp.bfloat16),
            pltpu.VMEM((m_out, n), jnp.float32),
            pltpu.SemaphoreType.DMA((2,)),
            pltpu.SemaphoreType.DMA((6,)),
            pltpu.SemaphoreType.DMA((6,)),
        ],
        compiler_params=pltpu.CompilerParams(collective_id=0),
    )(x, w_mat)


# device time: 17204 ns/iter; 1.0120x vs baseline; 1.0120x over previous
import jax
import jax.numpy as jnp
from jax import lax
from jax.experimental import pallas as pl
from jax.experimental.pallas import tpu as pltpu

N_DEV = 4

A1, A2, A3, B1, B2, B3 = range(6)


def kernel(x, w_mat):
    m, k_per = x.shape
    _, n = w_mat.shape
    m_out = m // N_DEV
    half = n // 2

    col_a = pl.ds(0, half)
    col_b = pl.ds(half, half)

    def body(x_ref, w_ref, out_ref, partial_ref, w_bf,
             send_ref, recv_ref, send_sems, recv_sems):
        my = lax.axis_index("i")
        left = lax.rem(my + (N_DEV - 1), N_DEV)
        right = lax.rem(my + 1, N_DEV)

        barrier_sem = pltpu.get_barrier_semaphore()
        for nbr in (left, right):
            pl.semaphore_signal(
                barrier_sem, inc=1,
                device_id=(nbr,), device_id_type=pl.DeviceIdType.MESH,
            )

        w_bf[...] = w_ref[...].astype(jnp.bfloat16)

        def rows(c):
            return pl.ds(lax.rem(my + c, N_DEV) * m_out, m_out)

        def gemm_chunk(c):
            partial_ref[rows(c), :] = jnp.dot(
                x_ref[rows(c), :].astype(jnp.bfloat16), w_bf[...],
                preferred_element_type=jnp.float32,
            )

        msgs = {}

        def send(i, tile, tgt):
            send_ref[i, :, :] = tile.astype(jnp.bfloat16)
            msgs[i] = pltpu.make_async_remote_copy(
                src_ref=send_ref.at[i],
                dst_ref=recv_ref.at[i],
                send_sem=send_sems.at[i],
                recv_sem=recv_sems.at[i],
                device_id=(tgt,),
                device_id_type=pl.DeviceIdType.MESH,
            )
            msgs[i].start()

        gemm_chunk(2)
        pl.semaphore_wait(barrier_sem, 2)
        send(A2, partial_ref[rows(2), col_a], left)
        send(B2, partial_ref[rows(2), col_b], right)

        gemm_chunk(1)
        send(A1, partial_ref[rows(1), col_a], right)
        gemm_chunk(3)
        send(B1, partial_ref[rows(3), col_b], left)
        gemm_chunk(0)

        msgs[A2].wait_recv()
        send(A3, recv_ref[A2].astype(jnp.float32)
             + partial_ref[rows(3), col_a], left)

        msgs[B2].wait_recv()
        send(B3, recv_ref[B2].astype(jnp.float32)
             + partial_ref[rows(1), col_b], right)

        msgs[A1].wait_recv()
        msgs[A3].wait_recv()
        y = (
            partial_ref[rows(0), col_a]
            + recv_ref[A1].astype(jnp.float32)
            + recv_ref[A3].astype(jnp.float32)
        )
        out_ref[:, col_a] = y * jax.nn.sigmoid(y)

        msgs[B1].wait_recv()
        msgs[B3].wait_recv()
        y = (
            partial_ref[rows(0), col_b]
            + recv_ref[B1].astype(jnp.float32)
            + recv_ref[B3].astype(jnp.float32)
        )
        out_ref[:, col_b] = y * jax.nn.sigmoid(y)

        for r in msgs.values():
            r.wait_send()

    return pl.pallas_call(
        body,
        out_shape=jax.ShapeDtypeStruct((m_out, n), jnp.float32),
        in_specs=[
            pl.BlockSpec(memory_space=pltpu.VMEM),
            pl.BlockSpec(memory_space=pltpu.VMEM),
        ],
        out_specs=pl.BlockSpec(memory_space=pltpu.VMEM),
        scratch_shapes=[
            pltpu.VMEM((m, n), jnp.float32),
            pltpu.VMEM((k_per, n), jnp.bfloat16),
            pltpu.VMEM((6, m_out, half), jnp.bfloat16),
            pltpu.VMEM((6, m_out, half), jnp.bfloat16),
            pltpu.SemaphoreType.DMA((6,)),
            pltpu.SemaphoreType.DMA((6,)),
        ],
        compiler_params=pltpu.CompilerParams(collective_id=0),
    )(x, w_mat)
